# Initial kernel scaffold; baseline (speedup 1.0000x reference)
import jax
import jax.numpy as jnp
from jax.experimental import pallas as pl


def kernel(x_paper, x_author, edge_cites, edge_writes, edge_rev_writes, params):
    raise NotImplementedError("write your pallas kernel here")


# speedup vs baseline: 1.2532x; 1.2532x over previous
import functools
import math

import jax
import jax.numpy as jnp
from jax.experimental import pallas as pl
from jax.experimental.pallas import tpu as pltpu

H = 4
HID = 128
D = HID // H


def _dense_stage(x, w_in, b_in, w_big, b_big):
    n = x.shape[0]
    kbig = w_big.shape[1]
    tile = 1000
    grid = (n // tile,)

    def body(x_ref, win_ref, bin_ref, wbig_ref, bbig_ref, h_ref, o_ref):
        h = jnp.maximum(
            jnp.dot(x_ref[...], win_ref[...], preferred_element_type=jnp.float32)
            + bin_ref[...], 0.0)
        h_ref[...] = h
        o_ref[...] = (jnp.dot(h, wbig_ref[...], preferred_element_type=jnp.float32)
                      + bbig_ref[...])

    return pl.pallas_call(
        body,
        grid=grid,
        in_specs=[
            pl.BlockSpec((tile, 128), lambda i: (i, 0)),
            pl.BlockSpec((128, HID), lambda i: (0, 0)),
            pl.BlockSpec((1, HID), lambda i: (0, 0)),
            pl.BlockSpec((HID, kbig), lambda i: (0, 0)),
            pl.BlockSpec((1, kbig), lambda i: (0, 0)),
        ],
        out_specs=[
            pl.BlockSpec((tile, HID), lambda i: (i, 0)),
            pl.BlockSpec((tile, kbig), lambda i: (i, 0)),
        ],
        out_shape=[
            jax.ShapeDtypeStruct((n, HID), jnp.float32),
            jax.ShapeDtypeStruct((n, kbig), jnp.float32),
        ],
    )(x, w_in, b_in.reshape(1, HID), w_big, b_big.reshape(1, kbig))


def _final_stage(agg, h, w_out, b_out, beta):
    n = agg.shape[0]
    tile = 1000
    grid = (n // tile,)

    def body(agg_ref, h_ref, w_ref, b_ref, beta_ref, o_ref):
        g = jax.nn.gelu(agg_ref[...])
        o = jnp.dot(g, w_ref[...], preferred_element_type=jnp.float32) + b_ref[...]
        bt = beta_ref[0, 0]
        o_ref[...] = bt * o + (1.0 - bt) * h_ref[...]

    return pl.pallas_call(
        body,
        grid=grid,
        in_specs=[
            pl.BlockSpec((tile, HID), lambda i: (i, 0)),
            pl.BlockSpec((tile, HID), lambda i: (i, 0)),
            pl.BlockSpec((HID, HID), lambda i: (0, 0)),
            pl.BlockSpec((1, HID), lambda i: (0, 0)),
            pl.BlockSpec((1, 1), lambda i: (0, 0), memory_space=pltpu.SMEM),
        ],
        out_specs=pl.BlockSpec((tile, HID), lambda i: (i, 0)),
        out_shape=jax.ShapeDtypeStruct((n, HID), jnp.float32),
    )(agg, h, w_out, b_out.reshape(1, HID), beta.reshape(1, 1))


def _edge_logits(q_dst, k_rel, v_rel, ei, p_et):
    src, dst = ei[0], ei[1]
    q_e = jnp.take(q_dst, dst, axis=0).reshape(-1, H, D)
    k_e = jnp.take(k_rel, src, axis=0).reshape(-1, H, D)
    v_e = jnp.take(v_rel, src, axis=0).reshape(-1, H, D)
    logit = (q_e * k_e).sum(-1) * p_et / math.sqrt(D)
    return logit, v_e, dst


def _aggregate(logits, vals, dst, n):
    m = jax.ops.segment_max(logits, dst, num_segments=n)
    m = jnp.where(jnp.isfinite(m), m, 0.0)
    a = jnp.exp(logits - jnp.take(m, dst, axis=0))
    s = jax.ops.segment_sum(a, dst, num_segments=n)
    alpha = a / (jnp.take(s, dst, axis=0) + 1e-16)
    out = jax.ops.segment_sum(alpha[..., None] * vals, dst, num_segments=n)
    return out.reshape(n, H * D)


def kernel(x_paper, x_author, edge_cites, edge_writes, edge_rev_writes, params):
    p = params

    def blockdiag(w):
        out = jnp.zeros((HID, HID), jnp.float32)
        for hh in range(H):
            out = out.at[hh * D:(hh + 1) * D, hh * D:(hh + 1) * D].set(w[hh])
        return out

    big_w = {}
    big_b = {}
    for nt, rels in (("paper", ("cites", "rev")), ("author", ("writes",))):
        wk = p[f"W_kqv_{nt}"][:, :HID]
        wq = p[f"W_kqv_{nt}"][:, HID:2 * HID]
        wv = p[f"W_kqv_{nt}"][:, 2 * HID:]
        bk = p[f"b_kqv_{nt}"][:HID]
        bq = p[f"b_kqv_{nt}"][HID:2 * HID]
        bv = p[f"b_kqv_{nt}"][2 * HID:]
        cols_w = [wq]
        cols_b = [bq]
        for et in rels:
            bdk = blockdiag(p[f"Wk_{et}"])
            bdv = blockdiag(p[f"Wv_{et}"])
            cols_w += [wk @ bdk, wv @ bdv]
            cols_b += [bk @ bdk, bv @ bdv]
        big_w[nt] = jnp.concatenate(cols_w, axis=1)
        big_b[nt] = jnp.concatenate(cols_b, axis=0)

    h_paper, big_paper = _dense_stage(
        x_paper, p["W_in_paper"], p["b_in_paper"], big_w["paper"], big_b["paper"])
    h_author, big_author = _dense_stage(
        x_author, p["W_in_author"], p["b_in_author"], big_w["author"], big_b["author"])

    q_paper = big_paper[:, :HID]
    krel_c = big_paper[:, HID:2 * HID]
    vrel_c = big_paper[:, 2 * HID:3 * HID]
    krel_r = big_paper[:, 3 * HID:4 * HID]
    vrel_r = big_paper[:, 4 * HID:5 * HID]
    q_author = big_author[:, :HID]
    krel_w = big_author[:, HID:2 * HID]
    vrel_w = big_author[:, 2 * HID:3 * HID]

    l1, v1, d1 = _edge_logits(q_paper, krel_c, vrel_c, edge_cites, p["p_cites"])
    l2, v2, d2 = _edge_logits(q_paper, krel_w, vrel_w, edge_writes, p["p_writes"])
    l3, v3, d3 = _edge_logits(q_author, krel_r, vrel_r, edge_rev_writes, p["p_rev"])

    agg_paper = _aggregate(jnp.concatenate([l1, l2], 0), jnp.concatenate([v1, v2], 0),
                           jnp.concatenate([d1, d2], 0), x_paper.shape[0])
    agg_author = _aggregate(l3, v3, d3, x_author.shape[0])

    out_paper = _final_stage(agg_paper, h_paper, p["W_out_paper"], p["b_out_paper"],
                             jax.nn.sigmoid(p["skip_paper"]))
    out_author = _final_stage(agg_author, h_author, p["W_out_author"], p["b_out_author"],
                              jax.nn.sigmoid(p["skip_author"]))
    return (out_paper, out_author)


# speedup vs baseline: 15.0401x; 12.0017x over previous
import functools
import math

import jax
from jax import lax
import jax.numpy as jnp
from jax.experimental import pallas as pl
from jax.experimental.pallas import tpu as pltpu
from jax.experimental.pallas import tpu_sc as plsc

H = 4
HID = 128
D = HID // H

_NC = 2
_NS = 16
_NW = _NC * _NS
_CHUNK = 128


def _sc_gather3(qtab, ktab, vtab, dsti, srci):
    e = dsti.shape[0]
    epad = -(-e // (_NW * _CHUNK)) * (_NW * _CHUNK)
    if epad != e:
        pad = jnp.zeros((epad - e,), jnp.int32)
        dsti = jnp.concatenate([dsti, pad])
        srci = jnp.concatenate([srci, pad])
    e_per_w = epad // _NW
    n_it = e_per_w // _CHUNK
    mesh = plsc.VectorSubcoreMesh(core_axis_name="c", subcore_axis_name="s")

    @functools.partial(
        pl.kernel,
        out_type=[jax.ShapeDtypeStruct((epad, HID), jnp.float32)] * 3,
        mesh=mesh,
        scratch_types=[
            pltpu.VMEM((_CHUNK,), jnp.int32),
            pltpu.VMEM((_CHUNK,), jnp.int32),
            pltpu.VMEM((_CHUNK, HID), jnp.float32),
            pltpu.VMEM((_CHUNK, HID), jnp.float32),
            pltpu.VMEM((_CHUNK, HID), jnp.float32),
            pltpu.SemaphoreType.DMA,
        ],
    )
    def k(qtab_h, ktab_h, vtab_h, dsti_h, srci_h, qe_h, ke_h, ve_h,
          di_v, si_v, qb, kb, vb, sem):
        wid = lax.axis_index("s") * _NC + lax.axis_index("c")
        w0 = wid * e_per_w

        def body(it, carry):
            base = pl.multiple_of(w0 + it * _CHUNK, _CHUNK)
            pltpu.sync_copy(dsti_h.at[pl.ds(base, _CHUNK)], di_v)
            pltpu.sync_copy(srci_h.at[pl.ds(base, _CHUNK)], si_v)
            c1 = pltpu.async_copy(qtab_h.at[di_v], qb, sem)
            c2 = pltpu.async_copy(ktab_h.at[si_v], kb, sem)
            c3 = pltpu.async_copy(vtab_h.at[si_v], vb, sem)
            c1.wait()
            c2.wait()
            c3.wait()
            pltpu.sync_copy(qb, qe_h.at[pl.ds(base, _CHUNK)])
            pltpu.sync_copy(kb, ke_h.at[pl.ds(base, _CHUNK)])
            pltpu.sync_copy(vb, ve_h.at[pl.ds(base, _CHUNK)])
            return carry

        lax.fori_loop(0, n_it, body, 0)

    return k(qtab, ktab, vtab, dsti, srci)


def _edge_math(qe, ke, ve, p_et):
    epad = qe.shape[0]
    tile = 512
    grid = (epad // tile,)
    scale = 1.0 / math.sqrt(D)

    def body(p_ref, q_ref, k_ref, v_ref, av_ref, ab_ref):
        qk = q_ref[...] * k_ref[...]
        for h in range(H):
            sl = slice(h * D, (h + 1) * D)
            logit = jnp.sum(qk[:, sl], axis=1, keepdims=True) * (
                p_ref[0, h] * scale)
            a = jnp.exp(logit)
            av_ref[:, sl] = v_ref[:, sl] * a
            ab_ref[:, sl] = jnp.broadcast_to(a, (tile, D))

    return pl.pallas_call(
        body,
        grid=grid,
        in_specs=[
            pl.BlockSpec((1, H), lambda i: (0, 0), memory_space=pltpu.SMEM),
            pl.BlockSpec((tile, HID), lambda i: (i, 0)),
            pl.BlockSpec((tile, HID), lambda i: (i, 0)),
            pl.BlockSpec((tile, HID), lambda i: (i, 0)),
        ],
        out_specs=[
            pl.BlockSpec((tile, HID), lambda i: (i, 0)),
            pl.BlockSpec((tile, HID), lambda i: (i, 0)),
        ],
        out_shape=[
            jax.ShapeDtypeStruct((epad, HID), jnp.float32),
            jax.ShapeDtypeStruct((epad, HID), jnp.float32),
        ],
    )(p_et.reshape(1, H), qe, ke, ve)


def _dense_stage(x, w_in, b_in, w_big, b_big):
    n = x.shape[0]
    kbig = w_big.shape[1]
    tile = 1000
    grid = (n // tile,)

    def body(x_ref, win_ref, bin_ref, wbig_ref, bbig_ref, h_ref, o_ref):
        h = jnp.maximum(
            jnp.dot(x_ref[...], win_ref[...], preferred_element_type=jnp.float32)
            + bin_ref[...], 0.0)
        h_ref[...] = h
        o_ref[...] = (jnp.dot(h, wbig_ref[...], preferred_element_type=jnp.float32)
                      + bbig_ref[...])

    return pl.pallas_call(
        body,
        grid=grid,
        in_specs=[
            pl.BlockSpec((tile, 128), lambda i: (i, 0)),
            pl.BlockSpec((128, HID), lambda i: (0, 0)),
            pl.BlockSpec((1, HID), lambda i: (0, 0)),
            pl.BlockSpec((HID, kbig), lambda i: (0, 0)),
            pl.BlockSpec((1, kbig), lambda i: (0, 0)),
        ],
        out_specs=[
            pl.BlockSpec((tile, HID), lambda i: (i, 0)),
            pl.BlockSpec((tile, kbig), lambda i: (i, 0)),
        ],
        out_shape=[
            jax.ShapeDtypeStruct((n, HID), jnp.float32),
            jax.ShapeDtypeStruct((n, kbig), jnp.float32),
        ],
    )(x, w_in, b_in.reshape(1, HID), w_big, b_big.reshape(1, kbig))


def _final_stage(num, den, h, w_out, b_out, beta):
    n = num.shape[0]
    tile = 1000
    grid = (n // tile,)

    def body(num_ref, den_ref, h_ref, w_ref, b_ref, beta_ref, o_ref):
        agg = num_ref[...] / (den_ref[...] + 1e-16)
        g = jax.nn.gelu(agg)
        o = jnp.dot(g, w_ref[...], preferred_element_type=jnp.float32) + b_ref[...]
        bt = beta_ref[0, 0]
        o_ref[...] = bt * o + (1.0 - bt) * h_ref[...]

    return pl.pallas_call(
        body,
        grid=grid,
        in_specs=[
            pl.BlockSpec((tile, HID), lambda i: (i, 0)),
            pl.BlockSpec((tile, HID), lambda i: (i, 0)),
            pl.BlockSpec((tile, HID), lambda i: (i, 0)),
            pl.BlockSpec((HID, HID), lambda i: (0, 0)),
            pl.BlockSpec((1, HID), lambda i: (0, 0)),
            pl.BlockSpec((1, 1), lambda i: (0, 0), memory_space=pltpu.SMEM),
        ],
        out_specs=pl.BlockSpec((tile, HID), lambda i: (i, 0)),
        out_shape=jax.ShapeDtypeStruct((n, HID), jnp.float32),
    )(num, den, h, w_out, b_out.reshape(1, HID), beta.reshape(1, 1))


def _edge_stage(q_dst, k_rel, v_rel, ei, p_et):
    src, dst = ei[0], ei[1]
    e = dst.shape[0]
    qe, ke, ve = _sc_gather3(q_dst, k_rel, v_rel, dst, src)
    av, ab = _edge_math(qe, ke, ve, p_et)
    return av[:e], ab[:e], dst


def kernel(x_paper, x_author, edge_cites, edge_writes, edge_rev_writes, params):
    p = params

    def blockdiag(w):
        out = jnp.zeros((HID, HID), jnp.float32)
        for hh in range(H):
            out = out.at[hh * D:(hh + 1) * D, hh * D:(hh + 1) * D].set(w[hh])
        return out

    big_w = {}
    big_b = {}
    for nt, rels in (("paper", ("cites", "rev")), ("author", ("writes",))):
        wk = p[f"W_kqv_{nt}"][:, :HID]
        wq = p[f"W_kqv_{nt}"][:, HID:2 * HID]
        wv = p[f"W_kqv_{nt}"][:, 2 * HID:]
        bk = p[f"b_kqv_{nt}"][:HID]
        bq = p[f"b_kqv_{nt}"][HID:2 * HID]
        bv = p[f"b_kqv_{nt}"][2 * HID:]
        cols_w = [wq]
        cols_b = [bq]
        for et in rels:
            bdk = blockdiag(p[f"Wk_{et}"])
            bdv = blockdiag(p[f"Wv_{et}"])
            cols_w += [wk @ bdk, wv @ bdv]
            cols_b += [bk @ bdk, bv @ bdv]
        big_w[nt] = jnp.concatenate(cols_w, axis=1)
        big_b[nt] = jnp.concatenate(cols_b, axis=0)

    h_paper, big_paper = _dense_stage(
        x_paper, p["W_in_paper"], p["b_in_paper"], big_w["paper"], big_b["paper"])
    h_author, big_author = _dense_stage(
        x_author, p["W_in_author"], p["b_in_author"], big_w["author"], big_b["author"])

    q_paper = big_paper[:, :HID]
    krel_c = big_paper[:, HID:2 * HID]
    vrel_c = big_paper[:, 2 * HID:3 * HID]
    krel_r = big_paper[:, 3 * HID:4 * HID]
    vrel_r = big_paper[:, 4 * HID:5 * HID]
    q_author = big_author[:, :HID]
    krel_w = big_author[:, HID:2 * HID]
    vrel_w = big_author[:, 2 * HID:3 * HID]

    av1, ab1, d1 = _edge_stage(q_paper, krel_c, vrel_c, edge_cites, p["p_cites"])
    av2, ab2, d2 = _edge_stage(q_paper, krel_w, vrel_w, edge_writes, p["p_writes"])
    av3, ab3, d3 = _edge_stage(q_author, krel_r, vrel_r, edge_rev_writes, p["p_rev"])

    n_p, n_a = x_paper.shape[0], x_author.shape[0]
    num_p = (jax.ops.segment_sum(av1, d1, num_segments=n_p)
             + jax.ops.segment_sum(av2, d2, num_segments=n_p))
    den_p = (jax.ops.segment_sum(ab1, d1, num_segments=n_p)
             + jax.ops.segment_sum(ab2, d2, num_segments=n_p))
    num_a = jax.ops.segment_sum(av3, d3, num_segments=n_a)
    den_a = jax.ops.segment_sum(ab3, d3, num_segments=n_a)

    out_paper = _final_stage(num_p, den_p, h_paper, p["W_out_paper"],
                             p["b_out_paper"], jax.nn.sigmoid(p["skip_paper"]))
    out_author = _final_stage(num_a, den_a, h_author, p["W_out_author"],
                              p["b_out_author"], jax.nn.sigmoid(p["skip_author"]))
    return (out_paper, out_author)
